# Initial kernel scaffold; baseline (speedup 1.0000x reference)
import functools

import jax
import jax.numpy as jnp
from jax import lax
from jax.experimental import pallas as pl
from jax.experimental.pallas import tpu as pltpu
from jax.experimental.pallas import tpu_sc as plsc

NC = 2
NS = 16
NW = NC * NS
LANES = 16


def _sc_segment_partials(table, src2d, et2d, dst2d, zeros, *, N, D, R, n_chunks, C):
    rows_per_tile = N // NS
    mesh = plsc.VectorSubcoreMesh(core_axis_name="c", subcore_axis_name="s")

    @functools.partial(
        pl.kernel,
        out_type=jax.ShapeDtypeStruct((NC, N, D), jnp.float32),
        mesh=mesh,
        scratch_types=[
            pltpu.VMEM((n_chunks, C), jnp.int32),
            pltpu.VMEM((n_chunks, C), jnp.int32),
            pltpu.VMEM((n_chunks, C), jnp.int32),
            pltpu.VMEM((n_chunks, C), jnp.int32),
            pltpu.VMEM((C, D), jnp.float32),
            pltpu.VMEM_SHARED((N, D), jnp.float32),
            pltpu.SemaphoreType.DMA,
        ],
    )
    def sc_kernel(table_h, src_h, et_h, dst_h, zeros_h, out_h,
                  src_v, et_v, dst_v, idx_v, rows_v, agg_s, sem):
        c = lax.axis_index("c")
        s = lax.axis_index("s")
        wid = c * NS + s

        row0 = s * rows_per_tile
        pltpu.sync_copy(zeros_h.at[pl.ds(row0, rows_per_tile)],
                        agg_s.at[pl.ds(row0, rows_per_tile)])

        ch0 = wid * n_chunks
        pltpu.sync_copy(src_h.at[pl.ds(ch0, n_chunks)], src_v)
        pltpu.sync_copy(et_h.at[pl.ds(ch0, n_chunks)], et_v)
        pltpu.sync_copy(dst_h.at[pl.ds(ch0, n_chunks)], dst_v)

        def idx_body(j, carry):
            for k in range(C // LANES):
                sl = pl.ds(k * LANES, LANES)
                idx_v[j, sl] = src_v[j, sl] * R + et_v[j, sl]
            return carry
        lax.fori_loop(0, n_chunks, idx_body, 0)

        plsc.subcore_barrier()

        def chunk_body(j, carry):
            pltpu.async_copy(table_h.at[idx_v.at[j]], rows_v, sem).wait()
            pltpu.sync_copy(rows_v, agg_s.at[dst_v.at[j]], add=True)
            return carry
        lax.fori_loop(0, n_chunks, chunk_body, 0)

        plsc.subcore_barrier()

        pltpu.sync_copy(agg_s.at[pl.ds(row0, rows_per_tile)],
                        out_h.at[c, pl.ds(row0, rows_per_tile)])

    return sc_kernel(table, src2d, et2d, dst2d, zeros)


def _tc_table(x, w_flat, *, N, D, RD, BN=1000):
    def body(x_ref, w_ref, o_ref):
        o_ref[...] = jnp.dot(x_ref[...], w_ref[...],
                             preferred_element_type=jnp.float32)

    return pl.pallas_call(
        body,
        grid=(N // BN,),
        in_specs=[
            pl.BlockSpec((BN, D), lambda i: (i, 0)),
            pl.BlockSpec((D, RD), lambda i: (0, 0)),
        ],
        out_specs=pl.BlockSpec((BN, RD), lambda i: (i, 0)),
        out_shape=jax.ShapeDtypeStruct((N, RD), jnp.float32),
    )(x, w_flat)


def _tc_final(x, partials, W_self, b_rel, W1a, W1b, b1, W2a, W2b, b2, *, N, D, BN=1000):
    def body(x_ref, p_ref, ws_ref, brel_ref, w1a_ref, w1b_ref, b1_ref,
             w2a_ref, w2b_ref, b2_ref, o_ref):
        xb = x_ref[...]
        agg = p_ref[0] + p_ref[1]
        h = agg + jnp.dot(xb, ws_ref[...], preferred_element_type=jnp.float32)
        msg = jnp.tanh(h + brel_ref[...])
        mid = jnp.tanh(
            jnp.dot(xb, w1a_ref[...], preferred_element_type=jnp.float32)
            + jnp.dot(msg, w1b_ref[...], preferred_element_type=jnp.float32)
            + b1_ref[...])
        o_ref[...] = jnp.tanh(
            jnp.dot(xb, w2a_ref[...], preferred_element_type=jnp.float32)
            + jnp.dot(mid, w2b_ref[...], preferred_element_type=jnp.float32)
            + b2_ref[...])

    def full(shape):
        return pl.BlockSpec(shape, lambda i: tuple(0 for _ in shape))

    return pl.pallas_call(
        body,
        grid=(N // BN,),
        in_specs=[
            pl.BlockSpec((BN, D), lambda i: (i, 0)),
            pl.BlockSpec((NC, BN, D), lambda i: (0, i, 0)),
            full((D, D)),
            full((1, D)),
            full((D, 2 * D)),
            full((D, 2 * D)),
            full((1, 2 * D)),
            full((D, D)),
            full((2 * D, D)),
            full((1, D)),
        ],
        out_specs=pl.BlockSpec((BN, D), lambda i: (i, 0)),
        out_shape=jax.ShapeDtypeStruct((N, D), jnp.float32),
    )(x, partials, W_self, b_rel, W1a, W1b, b1, W2a, W2b, b2)


def kernel(x, W_rel, W_self, b_rel, W1, b1, W2, b2, edge_index, edges_type,
           nodes_type, is_block):
    N, D = x.shape
    R = W_rel.shape[0]
    E = edges_type.shape[0]
    C = 80
    n_chunks = E // (NW * C)

    w_flat = jnp.transpose(W_rel, (1, 0, 2)).reshape(D, R * D)
    table = _tc_table(x, w_flat, N=N, D=D, RD=R * D).reshape(N * R, D)

    src2d = edge_index[0].reshape(NW * n_chunks, C)
    dst2d = edge_index[1].reshape(NW * n_chunks, C)
    et2d = edges_type.reshape(NW * n_chunks, C)
    zeros = jnp.zeros((N, D), jnp.float32)

    partials = _sc_segment_partials(table, src2d, et2d, dst2d, zeros,
                                    N=N, D=D, R=R, n_chunks=n_chunks, C=C)

    return _tc_final(
        x, partials, W_self, b_rel.reshape(1, D),
        W1[:D], W1[D:], b1.reshape(1, 2 * D),
        W2[:D], W2[D:], b2.reshape(1, D),
        N=N, D=D)


# speedup vs baseline: 20.0542x; 20.0542x over previous
import functools

import jax
import jax.numpy as jnp
from jax import lax
from jax.experimental import pallas as pl
from jax.experimental.pallas import tpu as pltpu
from jax.experimental.pallas import tpu_sc as plsc

NC = 2
NS = 16
NW = NC * NS
LANES = 16


def _sc_segment_partials(table, src4d, et4d, dst4d, zeros, *, N, D, R,
                         n_stages, n_stage, C):
    rows_per_tile = (N // NS) // 8 * 8
    tail_rows = N - rows_per_tile * NS
    mesh = plsc.VectorSubcoreMesh(core_axis_name="c", subcore_axis_name="s")

    @functools.partial(
        pl.kernel,
        out_type=jax.ShapeDtypeStruct((NC, N, D), jnp.float32),
        mesh=mesh,
        scratch_types=[
            pltpu.VMEM((n_stage, C), jnp.int32),
            pltpu.VMEM((n_stage, C), jnp.int32),
            pltpu.VMEM((n_stage, C), jnp.int32),
            pltpu.VMEM((C, D), jnp.float32),
            pltpu.VMEM_SHARED((N, D), jnp.float32),
            pltpu.SemaphoreType.DMA,
        ],
    )
    def sc_kernel(table_h, src_h, et_h, dst_h, zeros_h, out_h,
                  idx_v, et_v, dst_v, rows_v, agg_s, sem):
        c = lax.axis_index("c")
        s = lax.axis_index("s")
        wid = c * NS + s

        row0 = s * rows_per_tile
        pltpu.sync_copy(zeros_h.at[pl.ds(row0, rows_per_tile)],
                        agg_s.at[pl.ds(row0, rows_per_tile)])
        if tail_rows:
            @pl.when(s == NS - 1)
            def _():
                pltpu.sync_copy(zeros_h.at[pl.ds(NS * rows_per_tile, tail_rows)],
                                agg_s.at[pl.ds(NS * rows_per_tile, tail_rows)])

        plsc.subcore_barrier()

        def stage_body(st, carry):
            pltpu.sync_copy(src_h.at[wid, st], idx_v)
            pltpu.sync_copy(et_h.at[wid, st], et_v)
            pltpu.sync_copy(dst_h.at[wid, st], dst_v)

            def idx_body(j, carry2):
                for k in range(C // LANES):
                    sl = pl.ds(k * LANES, LANES)
                    idx_v[j, sl] = idx_v[j, sl] * R + et_v[j, sl]
                return carry2
            lax.fori_loop(0, n_stage, idx_body, 0)

            def chunk_body(j, carry2):
                pltpu.async_copy(table_h.at[idx_v.at[j]], rows_v, sem).wait()
                pltpu.sync_copy(rows_v, agg_s.at[dst_v.at[j]], add=True)
                return carry2
            lax.fori_loop(0, n_stage, chunk_body, 0)
            return carry
        lax.fori_loop(0, n_stages, stage_body, 0)

        plsc.subcore_barrier()

        pltpu.sync_copy(agg_s.at[pl.ds(row0, rows_per_tile)],
                        out_h.at[c, pl.ds(row0, rows_per_tile)])
        if tail_rows:
            @pl.when(s == NS - 1)
            def _():
                pltpu.sync_copy(agg_s.at[pl.ds(NS * rows_per_tile, tail_rows)],
                                out_h.at[c, pl.ds(NS * rows_per_tile, tail_rows)])

    return sc_kernel(table, src4d, et4d, dst4d, zeros)


def _tc_table(x, w_flat, *, N, D, RD, BN=1000):
    def body(x_ref, w_ref, o_ref):
        o_ref[...] = jnp.dot(x_ref[...], w_ref[...],
                             preferred_element_type=jnp.float32)

    return pl.pallas_call(
        body,
        grid=(N // BN,),
        in_specs=[
            pl.BlockSpec((BN, D), lambda i: (i, 0)),
            pl.BlockSpec((D, RD), lambda i: (0, 0)),
        ],
        out_specs=pl.BlockSpec((BN, RD), lambda i: (i, 0)),
        out_shape=jax.ShapeDtypeStruct((N, RD), jnp.float32),
    )(x, w_flat)


def _tc_final(x, partials, W_self, b_rel, W1a, W1b, b1, W2a, W2b, b2, *, N, D, BN=1000):
    def body(x_ref, p_ref, ws_ref, brel_ref, w1a_ref, w1b_ref, b1_ref,
             w2a_ref, w2b_ref, b2_ref, o_ref):
        xb = x_ref[...]
        agg = p_ref[0] + p_ref[1]
        h = agg + jnp.dot(xb, ws_ref[...], preferred_element_type=jnp.float32)
        msg = jnp.tanh(h + brel_ref[...])
        mid = jnp.tanh(
            jnp.dot(xb, w1a_ref[...], preferred_element_type=jnp.float32)
            + jnp.dot(msg, w1b_ref[...], preferred_element_type=jnp.float32)
            + b1_ref[...])
        o_ref[...] = jnp.tanh(
            jnp.dot(xb, w2a_ref[...], preferred_element_type=jnp.float32)
            + jnp.dot(mid, w2b_ref[...], preferred_element_type=jnp.float32)
            + b2_ref[...])

    def full(shape):
        return pl.BlockSpec(shape, lambda i: tuple(0 for _ in shape))

    return pl.pallas_call(
        body,
        grid=(N // BN,),
        in_specs=[
            pl.BlockSpec((BN, D), lambda i: (i, 0)),
            pl.BlockSpec((NC, BN, D), lambda i: (0, i, 0)),
            full((D, D)),
            full((1, D)),
            full((D, 2 * D)),
            full((D, 2 * D)),
            full((1, 2 * D)),
            full((D, D)),
            full((2 * D, D)),
            full((1, D)),
        ],
        out_specs=pl.BlockSpec((BN, D), lambda i: (i, 0)),
        out_shape=jax.ShapeDtypeStruct((N, D), jnp.float32),
    )(x, partials, W_self, b_rel, W1a, W1b, b1, W2a, W2b, b2)


def kernel(x, W_rel, W_self, b_rel, W1, b1, W2, b2, edge_index, edges_type,
           nodes_type, is_block):
    N, D = x.shape
    R = W_rel.shape[0]
    E = edges_type.shape[0]
    C = 80
    n_chunks = E // (NW * C)
    n_stages = 5
    n_stage = n_chunks // n_stages

    w_flat = jnp.transpose(W_rel, (1, 0, 2)).reshape(D, R * D)
    table = _tc_table(x, w_flat, N=N, D=D, RD=R * D).reshape(N * R, D)

    src4d = edge_index[0].reshape(NW, n_stages, n_stage, C)
    dst4d = edge_index[1].reshape(NW, n_stages, n_stage, C)
    et4d = edges_type.reshape(NW, n_stages, n_stage, C)
    zeros = jnp.zeros((N, D), jnp.float32)

    partials = _sc_segment_partials(table, src4d, et4d, dst4d, zeros,
                                    N=N, D=D, R=R, n_stages=n_stages,
                                    n_stage=n_stage, C=C)

    return _tc_final(
        x, partials, W_self, b_rel.reshape(1, D),
        W1[:D], W1[D:], b1.reshape(1, 2 * D),
        W2[:D], W2[D:], b2.reshape(1, D),
        N=N, D=D)


# speedup vs baseline: 26.8245x; 1.3376x over previous
import functools

import jax
import jax.numpy as jnp
from jax import lax
from jax.experimental import pallas as pl
from jax.experimental.pallas import tpu as pltpu
from jax.experimental.pallas import tpu_sc as plsc

NC = 2
NS = 16
NW = NC * NS
LANES = 16


def _sc_segment_partials(table, src4d, et4d, dst4d, zeros, *, N, D, R,
                         n_stages, n_stage, C):
    rows_per_tile = (N // NS) // 8 * 8
    tail_rows = N - rows_per_tile * NS
    mesh = plsc.VectorSubcoreMesh(core_axis_name="c", subcore_axis_name="s")

    @functools.partial(
        pl.kernel,
        out_type=jax.ShapeDtypeStruct((NC, N, D), jnp.float32),
        mesh=mesh,
        scratch_types=[
            pltpu.VMEM((n_stage, C), jnp.int32),
            pltpu.VMEM((n_stage, C), jnp.int32),
            pltpu.VMEM((n_stage, C), jnp.int32),
            pltpu.VMEM((C, D), jnp.float32),
            pltpu.VMEM((C, D), jnp.float32),
            pltpu.VMEM_SHARED((N, D), jnp.float32),
            pltpu.SemaphoreType.DMA,
            pltpu.SemaphoreType.DMA,
        ],
    )
    def sc_kernel(table_h, src_h, et_h, dst_h, zeros_h, out_h,
                  idx_v, et_v, dst_v, rows0_v, rows1_v, agg_s, sem0, sem1):
        c = lax.axis_index("c")
        s = lax.axis_index("s")
        wid = c * NS + s

        row0 = s * rows_per_tile
        pltpu.sync_copy(zeros_h.at[pl.ds(row0, rows_per_tile)],
                        agg_s.at[pl.ds(row0, rows_per_tile)])
        if tail_rows:
            @pl.when(s == NS - 1)
            def _():
                pltpu.sync_copy(zeros_h.at[pl.ds(NS * rows_per_tile, tail_rows)],
                                agg_s.at[pl.ds(NS * rows_per_tile, tail_rows)])

        plsc.subcore_barrier()

        def stage_body(st, carry):
            pltpu.sync_copy(src_h.at[wid, st], idx_v)
            pltpu.sync_copy(et_h.at[wid, st], et_v)
            pltpu.sync_copy(dst_h.at[wid, st], dst_v)

            def idx_body(j, carry2):
                for k in range(C // LANES):
                    sl = pl.ds(k * LANES, LANES)
                    idx_v[j, sl] = idx_v[j, sl] * R + et_v[j, sl]
                return carry2
            lax.fori_loop(0, n_stage, idx_body, 0)

            rows = (rows0_v, rows1_v)
            sems = (sem0, sem1)
            descs = [None, None]
            descs[0] = pltpu.async_copy(table_h.at[idx_v.at[0]], rows[0], sems[0])
            for j in range(n_stage):
                b = j & 1
                if j + 1 < n_stage:
                    descs[1 - b] = pltpu.async_copy(
                        table_h.at[idx_v.at[j + 1]], rows[1 - b], sems[1 - b])
                descs[b].wait()
                pltpu.sync_copy(rows[b], agg_s.at[dst_v.at[j]], add=True)
            return carry
        lax.fori_loop(0, n_stages, stage_body, 0)

        plsc.subcore_barrier()

        pltpu.sync_copy(agg_s.at[pl.ds(row0, rows_per_tile)],
                        out_h.at[c, pl.ds(row0, rows_per_tile)])
        if tail_rows:
            @pl.when(s == NS - 1)
            def _():
                pltpu.sync_copy(agg_s.at[pl.ds(NS * rows_per_tile, tail_rows)],
                                out_h.at[c, pl.ds(NS * rows_per_tile, tail_rows)])

    return sc_kernel(table, src4d, et4d, dst4d, zeros)


def _tc_table(x, w_flat, *, N, D, RD, BN=1000):
    def body(x_ref, w_ref, o_ref):
        o_ref[...] = jnp.dot(x_ref[...], w_ref[...],
                             preferred_element_type=jnp.float32)

    return pl.pallas_call(
        body,
        grid=(N // BN,),
        in_specs=[
            pl.BlockSpec((BN, D), lambda i: (i, 0)),
            pl.BlockSpec((D, RD), lambda i: (0, 0)),
        ],
        out_specs=pl.BlockSpec((BN, RD), lambda i: (i, 0)),
        out_shape=jax.ShapeDtypeStruct((N, RD), jnp.float32),
    )(x, w_flat)


def _tc_final(x, partials, W_self, b_rel, W1a, W1b, b1, W2a, W2b, b2, *, N, D, BN=1000):
    def body(x_ref, p_ref, ws_ref, brel_ref, w1a_ref, w1b_ref, b1_ref,
             w2a_ref, w2b_ref, b2_ref, o_ref):
        xb = x_ref[...]
        agg = p_ref[0] + p_ref[1]
        h = agg + jnp.dot(xb, ws_ref[...], preferred_element_type=jnp.float32)
        msg = jnp.tanh(h + brel_ref[...])
        mid = jnp.tanh(
            jnp.dot(xb, w1a_ref[...], preferred_element_type=jnp.float32)
            + jnp.dot(msg, w1b_ref[...], preferred_element_type=jnp.float32)
            + b1_ref[...])
        o_ref[...] = jnp.tanh(
            jnp.dot(xb, w2a_ref[...], preferred_element_type=jnp.float32)
            + jnp.dot(mid, w2b_ref[...], preferred_element_type=jnp.float32)
            + b2_ref[...])

    def full(shape):
        return pl.BlockSpec(shape, lambda i: tuple(0 for _ in shape))

    return pl.pallas_call(
        body,
        grid=(N // BN,),
        in_specs=[
            pl.BlockSpec((BN, D), lambda i: (i, 0)),
            pl.BlockSpec((NC, BN, D), lambda i: (0, i, 0)),
            full((D, D)),
            full((1, D)),
            full((D, 2 * D)),
            full((D, 2 * D)),
            full((1, 2 * D)),
            full((D, D)),
            full((2 * D, D)),
            full((1, D)),
        ],
        out_specs=pl.BlockSpec((BN, D), lambda i: (i, 0)),
        out_shape=jax.ShapeDtypeStruct((N, D), jnp.float32),
    )(x, partials, W_self, b_rel, W1a, W1b, b1, W2a, W2b, b2)


def kernel(x, W_rel, W_self, b_rel, W1, b1, W2, b2, edge_index, edges_type,
           nodes_type, is_block):
    N, D = x.shape
    R = W_rel.shape[0]
    E = edges_type.shape[0]
    C = 80
    n_chunks = E // (NW * C)
    n_stages = 5
    n_stage = n_chunks // n_stages

    w_flat = jnp.transpose(W_rel, (1, 0, 2)).reshape(D, R * D)
    table = _tc_table(x, w_flat, N=N, D=D, RD=R * D).reshape(N * R, D)

    src4d = edge_index[0].reshape(NW, n_stages, n_stage, C)
    dst4d = edge_index[1].reshape(NW, n_stages, n_stage, C)
    et4d = edges_type.reshape(NW, n_stages, n_stage, C)
    zeros = jnp.zeros((N, D), jnp.float32)

    partials = _sc_segment_partials(table, src4d, et4d, dst4d, zeros,
                                    N=N, D=D, R=R, n_stages=n_stages,
                                    n_stage=n_stage, C=C)

    return _tc_final(
        x, partials, W_self, b_rel.reshape(1, D),
        W1[:D], W1[D:], b1.reshape(1, 2 * D),
        W2[:D], W2[D:], b2.reshape(1, D),
        N=N, D=D)


# speedup vs baseline: 27.3244x; 1.0186x over previous
import functools

import jax
import jax.numpy as jnp
from jax import lax
from jax.experimental import pallas as pl
from jax.experimental.pallas import tpu as pltpu
from jax.experimental.pallas import tpu_sc as plsc

NC = 2
NS = 16
NW = NC * NS
LANES = 16


def _sc_segment_partials(table, epk, zeros, *, N, D, R, n_stages, n_stage, C):
    rows_per_tile = (N // NS) // 8 * 8
    tail_rows = N - rows_per_tile * NS
    mesh = plsc.VectorSubcoreMesh(core_axis_name="c", subcore_axis_name="s")

    @functools.partial(
        pl.kernel,
        out_type=jax.ShapeDtypeStruct((NC, N, D), jnp.float32),
        mesh=mesh,
        scratch_types=[
            pltpu.VMEM((3, n_stage, C), jnp.int32),
            pltpu.VMEM((C, D), jnp.float32),
            pltpu.VMEM((C, D), jnp.float32),
            pltpu.VMEM_SHARED((N, D), jnp.float32),
            pltpu.SemaphoreType.DMA,
            pltpu.SemaphoreType.DMA,
            pltpu.SemaphoreType.DMA,
        ],
    )
    def sc_kernel(table_h, epk_h, zeros_h, out_h,
                  e_v, rows0_v, rows1_v, agg_s, zsem, sem0, sem1):
        c = lax.axis_index("c")
        s = lax.axis_index("s")
        wid = c * NS + s
        row0 = s * rows_per_tile

        def stage_load(st):
            pltpu.sync_copy(epk_h.at[wid, st], e_v)

            def idx_body(j, carry2):
                for k in range(C // LANES):
                    sl = pl.ds(k * LANES, LANES)
                    e_v[0, j, sl] = e_v[0, j, sl] * R + e_v[1, j, sl]
                return carry2
            lax.fori_loop(0, n_stage, idx_body, 0)

        zdesc = pltpu.async_copy(zeros_h.at[pl.ds(row0, rows_per_tile)],
                                 agg_s.at[pl.ds(row0, rows_per_tile)], zsem)
        stage_load(0)
        zdesc.wait()
        if tail_rows:
            @pl.when(s == NS - 1)
            def _():
                pltpu.sync_copy(zeros_h.at[pl.ds(NS * rows_per_tile, tail_rows)],
                                agg_s.at[pl.ds(NS * rows_per_tile, tail_rows)])

        plsc.subcore_barrier()

        def stage_body(st, carry):
            rows = (rows0_v, rows1_v)
            sems = (sem0, sem1)
            descs = [None, None]
            descs[0] = pltpu.async_copy(table_h.at[e_v.at[0, 0]], rows[0], sems[0])
            for j in range(n_stage):
                b = j & 1
                if j + 1 < n_stage:
                    descs[1 - b] = pltpu.async_copy(
                        table_h.at[e_v.at[0, j + 1]], rows[1 - b], sems[1 - b])
                descs[b].wait()
                pltpu.sync_copy(rows[b], agg_s.at[e_v.at[2, j]], add=True)

            @pl.when(st + 1 < n_stages)
            def _():
                stage_load(st + 1)
            return carry
        lax.fori_loop(0, n_stages, stage_body, 0)

        plsc.subcore_barrier()

        pltpu.sync_copy(agg_s.at[pl.ds(row0, rows_per_tile)],
                        out_h.at[c, pl.ds(row0, rows_per_tile)])
        if tail_rows:
            @pl.when(s == NS - 1)
            def _():
                pltpu.sync_copy(agg_s.at[pl.ds(NS * rows_per_tile, tail_rows)],
                                out_h.at[c, pl.ds(NS * rows_per_tile, tail_rows)])

    return sc_kernel(table, epk, zeros)


def _tc_table(x, w_flat, *, N, D, RD, BN=2000):
    def body(x_ref, w_ref, o_ref):
        o_ref[...] = jnp.dot(x_ref[...], w_ref[...],
                             preferred_element_type=jnp.float32)

    return pl.pallas_call(
        body,
        grid=(N // BN,),
        in_specs=[
            pl.BlockSpec((BN, D), lambda i: (i, 0)),
            pl.BlockSpec((D, RD), lambda i: (0, 0)),
        ],
        out_specs=pl.BlockSpec((BN, RD), lambda i: (i, 0)),
        out_shape=jax.ShapeDtypeStruct((N, RD), jnp.float32),
    )(x, w_flat)


def _tc_final(x, partials, W_self, b_rel, W1a, W1b, b1, W2a, W2b, b2, *, N, D, BN=2000):
    def body(x_ref, p_ref, ws_ref, brel_ref, w1a_ref, w1b_ref, b1_ref,
             w2a_ref, w2b_ref, b2_ref, o_ref):
        xb = x_ref[...]
        agg = p_ref[0] + p_ref[1]
        h = agg + jnp.dot(xb, ws_ref[...], preferred_element_type=jnp.float32)
        msg = jnp.tanh(h + brel_ref[...])
        mid = jnp.tanh(
            jnp.dot(xb, w1a_ref[...], preferred_element_type=jnp.float32)
            + jnp.dot(msg, w1b_ref[...], preferred_element_type=jnp.float32)
            + b1_ref[...])
        o_ref[...] = jnp.tanh(
            jnp.dot(xb, w2a_ref[...], preferred_element_type=jnp.float32)
            + jnp.dot(mid, w2b_ref[...], preferred_element_type=jnp.float32)
            + b2_ref[...])

    def full(shape):
        return pl.BlockSpec(shape, lambda i: tuple(0 for _ in shape))

    return pl.pallas_call(
        body,
        grid=(N // BN,),
        in_specs=[
            pl.BlockSpec((BN, D), lambda i: (i, 0)),
            pl.BlockSpec((NC, BN, D), lambda i: (0, i, 0)),
            full((D, D)),
            full((1, D)),
            full((D, 2 * D)),
            full((D, 2 * D)),
            full((1, 2 * D)),
            full((D, D)),
            full((2 * D, D)),
            full((1, D)),
        ],
        out_specs=pl.BlockSpec((BN, D), lambda i: (i, 0)),
        out_shape=jax.ShapeDtypeStruct((N, D), jnp.float32),
    )(x, partials, W_self, b_rel, W1a, W1b, b1, W2a, W2b, b2)


def kernel(x, W_rel, W_self, b_rel, W1, b1, W2, b2, edge_index, edges_type,
           nodes_type, is_block):
    N, D = x.shape
    R = W_rel.shape[0]
    E = edges_type.shape[0]
    C = 80
    n_chunks = E // (NW * C)
    n_stages = 5
    n_stage = n_chunks // n_stages

    w_flat = jnp.transpose(W_rel, (1, 0, 2)).reshape(D, R * D)
    table = _tc_table(x, w_flat, N=N, D=D, RD=R * D).reshape(N * R, D)

    epk = jnp.stack([edge_index[0], edges_type, edge_index[1]])
    epk = epk.reshape(3, NW, n_stages, n_stage, C).transpose(1, 2, 0, 3, 4)
    zeros = jnp.zeros((N, D), jnp.float32)

    partials = _sc_segment_partials(table, epk, zeros,
                                    N=N, D=D, R=R, n_stages=n_stages,
                                    n_stage=n_stage, C=C)

    return _tc_final(
        x, partials, W_self, b_rel.reshape(1, D),
        W1[:D], W1[D:], b1.reshape(1, 2 * D),
        W2[:D], W2[D:], b2.reshape(1, D),
        N=N, D=D)
